# baseline (speedup 1.0000x reference)
import functools

import jax
import jax.numpy as jnp
from jax import lax
from jax.experimental import pallas as pl
from jax.experimental.pallas import tpu as pltpu
from jax.experimental.pallas import tpu_sc as plsc

N = 50000
B = 256
D = 256
L = 16
KS = D // L
NC = 2
NS = 16
NW = NC * NS
RB = 56
NBLK = N // RB
NFULL = NBLK * RB
TAIL = N - NFULL
BPW = (NBLK + NW - 1) // NW


def _sc_body(x_ref, bidx_ref, wg_ref, qp_ref, dp_ref,
             xbuf, idxbuf, wgbuf, accq, accd, semx, semi):
    cid = lax.axis_index("c")
    sid = lax.axis_index("s")
    wid = sid * NC + cid

    pltpu.sync_copy(wg_ref, wgbuf)

    zv = jnp.zeros((L,), jnp.float32)

    def _zero(r):
        for k in range(KS):
            accq[r, pl.ds(L * k, L)] = zv
        accd[r, :] = zv
    plsc.parallel_loop(0, B, unroll=8)(_zero)

    def _x_copy(blk, s):
        return pltpu.make_async_copy(
            x_ref.at[pl.ds(blk * RB, RB)], xbuf.at[s], semx.at[s])

    def _i_copy(blk, s):
        return pltpu.make_async_copy(
            bidx_ref.at[pl.ds(blk * RB, RB)], idxbuf.at[s, pl.ds(0, RB)],
            semi.at[s])

    _x_copy(wid, 0).start()
    _i_copy(wid, 0).start()

    def _proc(i, c):
        blk = wid + NW * i
        s = lax.rem(i, 2)
        _x_copy(blk, s).wait()
        _i_copy(blk, s).wait()
        idxbuf[s, pl.ds(RB, L)] = jnp.full((L,), -1, jnp.int32)

        nblk = blk + NW

        @pl.when(nblk < NBLK)
        def _():
            ns = lax.rem(i + 1, 2)
            _x_copy(nblk, ns).start()
            _i_copy(nblk, ns).start()

        def _run(r0):
            b = idxbuf[s, pl.ds(r0, L)][0]

            def _bs(_, lh):
                lo, hi = lh
                mid = lax.div(lo + hi, 2)
                same = idxbuf[s, pl.ds(mid, L)][0] == b
                return jnp.where(same, mid, lo), jnp.where(same, hi, mid)
            _, rend = lax.fori_loop(0, 6, _bs, (r0, jnp.int32(RB)))

            def _rowF(r, frc):
                rdv, rv = frc
                xs = [xbuf[s, r, pl.ds(L * k, L)] for k in range(KS)]
                prods = [xs[k] * wgbuf[pl.ds(L * k, L)] for k in range(KS)]
                while len(prods) > 1:
                    prods = [prods[j] + prods[j + 1]
                             for j in range(0, len(prods) - 1, 2)] + (
                                 [prods[-1]] if len(prods) % 2 else [])
                ev = jnp.exp(zv + jnp.sum(prods[0]))
                return rdv + ev, tuple(rv[k] + ev * xs[k] for k in range(KS))

            rd, racc = plsc.parallel_loop(
                r0, rend, unroll=4,
                carry=(zv, tuple(zv for _ in range(KS))))(_rowF)
            for k in range(KS):
                sl = pl.ds(L * k, L)
                accq[b, sl] = accq[b, sl] + racc[k]
            accd[b, :] = accd[b, :] + rd
            return rend
        lax.while_loop(lambda r0: r0 < RB, _run, jnp.int32(0))
        return c

    def _block(i, c):
        blk = wid + NW * i
        return lax.cond(blk < NBLK, _proc, lambda i_, c_: c_, i, c)

    lax.fori_loop(0, BPW, _block, 0)

    pltpu.sync_copy(accq, qp_ref.at[wid])
    pltpu.sync_copy(accd, dp_ref.at[wid])


_sc_pool = pl.kernel(
    _sc_body,
    out_type=(
        jax.ShapeDtypeStruct((NW, B, D), jnp.float32),
        jax.ShapeDtypeStruct((NW, B, L), jnp.float32),
    ),
    mesh=plsc.VectorSubcoreMesh(
        core_axis_name="c", subcore_axis_name="s",
        num_cores=NC, num_subcores=NS),
    compiler_params=pltpu.CompilerParams(needs_layout_passes=False),
    scratch_types=[
        pltpu.VMEM((2, RB, D), jnp.float32),
        pltpu.VMEM((2, RB + L), jnp.int32),
        pltpu.VMEM((D,), jnp.float32),
        pltpu.VMEM((B, D), jnp.float32),
        pltpu.VMEM((B, L), jnp.float32),
        pltpu.SemaphoreType.DMA((2,)),
        pltpu.SemaphoreType.DMA((2,)),
    ],
)


def _tc_body(qp, dp, xt, bt, gp, wg, bg, wnn, bnn, wl1, wl2, bl, out):
    q = jnp.sum(qp[...], axis=0)
    dcol = jnp.sum(dp[...], axis=0)[:, 0:1]

    xtail = xt[...]
    gate_t = jnp.dot(xtail, wg[...], preferred_element_type=jnp.float32)
    et = jnp.exp(gate_t)
    iota = lax.broadcasted_iota(jnp.int32, (TAIL, B), 1)
    oh = (bt[...] == iota).astype(jnp.float32)
    q = q + lax.dot_general(oh, et * xtail, (((0,), (0,)), ((), ())),
                            preferred_element_type=jnp.float32)
    dcol = dcol + lax.dot_general(oh, et, (((0,), (0,)), ((), ())),
                                  preferred_element_type=jnp.float32)

    ebg = jnp.exp(bg[0, 0])
    t = dcol * ebg
    p = q * (ebg / (t + 1e-16))
    s = t / (t + 1e-16)

    g = jnp.dot(p, wnn[...], preferred_element_type=jnp.float32) + s * bnn[...]
    h = (jnp.dot(g, wl1[...], preferred_element_type=jnp.float32)
         + jnp.dot(gp[...], wl2[...], preferred_element_type=jnp.float32)
         + bl[...])
    out[...] = gp[...] + jnp.maximum(h, 0.0)


_tc_call = pl.pallas_call(
    _tc_body,
    out_shape=jax.ShapeDtypeStruct((B, D), jnp.float32),
)


def kernel(x, g_prev, batch_idx, W_gate, b_gate, W_nn, b_nn, W_lin, b_lin):
    bidx = batch_idx.astype(jnp.int32)
    qp, dp = _sc_pool(x, bidx, W_gate.reshape(D))
    xtail = x[NFULL:]
    bt = jnp.broadcast_to(bidx[NFULL:, None], (TAIL, B))
    return _tc_call(qp, dp, xtail, bt, g_prev,
                    W_gate, b_gate.reshape(1, 1), W_nn,
                    b_nn.reshape(1, D), W_lin[:D], W_lin[D:],
                    b_lin.reshape(1, D))


# speedup vs baseline: 1.0165x; 1.0165x over previous
import jax
import jax.numpy as jnp
from jax import lax
from jax.experimental import pallas as pl
from jax.experimental.pallas import tpu as pltpu
from jax.experimental.pallas import tpu_sc as plsc

N = 50000
B = 256
D = 256
L = 16
KS = D // L
NC = 2
NS = 16
NW = NC * NS
RB = 56
NBLK = N // RB
NFULL = NBLK * RB
TAIL = N - NFULL
BPW = (NBLK + NW - 1) // NW


def _sc_body(x_ref, bidx_ref, wg_ref, qp_ref, dp_ref,
             xbuf, idxbuf, wgbuf, accq, accd, semx, semi):
    cid = lax.axis_index("c")
    sid = lax.axis_index("s")
    wid = sid * NC + cid

    pltpu.sync_copy(wg_ref, wgbuf)

    zv = jnp.zeros((L,), jnp.float32)

    def _zero(r):
        for k in range(KS):
            accq[r, pl.ds(L * k, L)] = zv
        accd[r, :] = zv
    plsc.parallel_loop(0, B, unroll=8)(_zero)

    def _x_copy(blk, s):
        return pltpu.make_async_copy(
            x_ref.at[pl.ds(blk * RB, RB)], xbuf.at[s], semx.at[s])

    def _i_copy(blk, s):
        return pltpu.make_async_copy(
            bidx_ref.at[pl.ds(blk * RB, RB)], idxbuf.at[s, pl.ds(0, RB)],
            semi.at[s])

    _x_copy(wid, 0).start()
    _i_copy(wid, 0).start()

    def _proc(i, c):
        blk = wid + NW * i
        s = lax.rem(i, 2)
        _x_copy(blk, s).wait()
        _i_copy(blk, s).wait()
        idxbuf[s, pl.ds(RB, L)] = jnp.full((L,), -1, jnp.int32)

        nblk = blk + NW

        @pl.when(nblk < NBLK)
        def _():
            ns = lax.rem(i + 1, 2)
            _x_copy(nblk, ns).start()
            _i_copy(nblk, ns).start()

        def _run(r0):
            b = idxbuf[s, pl.ds(r0, L)][0]

            def _bs(_, lh):
                lo, hi = lh
                mid = lax.div(lo + hi, 2)
                same = idxbuf[s, pl.ds(mid, L)][0] == b
                return jnp.where(same, mid, lo), jnp.where(same, hi, mid)
            _, rend = lax.fori_loop(0, 6, _bs, (r0, jnp.int32(RB)))

            def _rowF(r, frc):
                rdv, rv = frc
                xs = [xbuf[s, r, pl.ds(L * k, L)] for k in range(KS)]
                prods = [xs[k] * wgbuf[pl.ds(L * k, L)] for k in range(KS)]
                while len(prods) > 1:
                    prods = [prods[j] + prods[j + 1]
                             for j in range(0, len(prods) - 1, 2)] + (
                                 [prods[-1]] if len(prods) % 2 else [])
                ev = jnp.exp(zv + jnp.sum(prods[0]))
                return rdv + ev, tuple(rv[k] + ev * xs[k] for k in range(KS))

            rd, racc = plsc.parallel_loop(
                r0, rend, unroll=2,
                carry=(zv, tuple(zv for _ in range(KS))))(_rowF)
            for k in range(KS):
                sl = pl.ds(L * k, L)
                accq[b, sl] = accq[b, sl] + racc[k]
            accd[b, :] = accd[b, :] + rd
            return rend
        lax.while_loop(lambda r0: r0 < RB, _run, jnp.int32(0))
        return c

    def _block(i, c):
        blk = wid + NW * i
        return lax.cond(blk < NBLK, _proc, lambda i_, c_: c_, i, c)

    lax.fori_loop(0, BPW, _block, 0)

    pltpu.sync_copy(accq, qp_ref.at[wid])
    pltpu.sync_copy(accd, dp_ref.at[wid])


_sc_pool = pl.kernel(
    _sc_body,
    out_type=(
        jax.ShapeDtypeStruct((NW, B, D), jnp.float32),
        jax.ShapeDtypeStruct((NW, B, L), jnp.float32),
    ),
    mesh=plsc.VectorSubcoreMesh(
        core_axis_name="c", subcore_axis_name="s",
        num_cores=NC, num_subcores=NS),
    compiler_params=pltpu.CompilerParams(needs_layout_passes=False),
    scratch_types=[
        pltpu.VMEM((2, RB, D), jnp.float32),
        pltpu.VMEM((2, RB + L), jnp.int32),
        pltpu.VMEM((D,), jnp.float32),
        pltpu.VMEM((B, D), jnp.float32),
        pltpu.VMEM((B, L), jnp.float32),
        pltpu.SemaphoreType.DMA((2,)),
        pltpu.SemaphoreType.DMA((2,)),
    ],
)


def _tc_body(qp, dp, xt, bt, gp, wg, bg, wnn, bnn, wl1, wl2, bl, out):
    q = jnp.sum(qp[...], axis=0)
    dcol = jnp.sum(dp[...], axis=0)[:, 0:1]

    xtail = xt[...]
    gate_t = jnp.dot(xtail, wg[...], preferred_element_type=jnp.float32)
    et = jnp.exp(gate_t)
    iota = lax.broadcasted_iota(jnp.int32, (TAIL, B), 1)
    oh = (bt[...] == iota).astype(jnp.float32)
    q = q + lax.dot_general(oh, et * xtail, (((0,), (0,)), ((), ())),
                            preferred_element_type=jnp.float32)
    dcol = dcol + lax.dot_general(oh, et, (((0,), (0,)), ((), ())),
                                  preferred_element_type=jnp.float32)

    ebg = jnp.exp(bg[0, 0])
    t = dcol * ebg
    p = q * (ebg / (t + 1e-16))
    s = t / (t + 1e-16)

    g = jnp.dot(p, wnn[...], preferred_element_type=jnp.float32) + s * bnn[...]
    h = (jnp.dot(g, wl1[...], preferred_element_type=jnp.float32)
         + jnp.dot(gp[...], wl2[...], preferred_element_type=jnp.float32)
         + bl[...])
    out[...] = gp[...] + jnp.maximum(h, 0.0)


_tc_call = pl.pallas_call(
    _tc_body,
    out_shape=jax.ShapeDtypeStruct((B, D), jnp.float32),
)


def kernel(x, g_prev, batch_idx, W_gate, b_gate, W_nn, b_nn, W_lin, b_lin):
    bidx = batch_idx.astype(jnp.int32)
    qp, dp = _sc_pool(x, bidx, W_gate.reshape(D))
    xtail = x[NFULL:]
    bt = jnp.broadcast_to(bidx[NFULL:, None], (TAIL, B))
    return _tc_call(qp, dp, xtail, bt, g_prev,
                    W_gate, b_gate.reshape(1, 1), W_nn,
                    b_nn.reshape(1, D), W_lin[:D], W_lin[D:],
                    b_lin.reshape(1, D))
